# baseline (speedup 1.0000x reference)
import jax
import jax.numpy as jnp
from jax.experimental import pallas as pl
from jax.experimental.pallas import tpu as pltpu

_B, _E, _H, _W = 32, 768, 16, 16
_HW = _H * _W
_GRP = 4
_NSEM = 8


def _tc_body(row_ref, col_ref, o_hbm, img, sems):
    row16 = row_ref[...]
    col16 = col_ref[...]
    pos2 = jnp.repeat(row16, _W, axis=0) + jnp.tile(col16, (_H, 1))
    groups = [(0, 1), (1, 1)] + [(b, 2) for b in range(2, _B, 2)]
    for k, (b0, n) in enumerate(groups):
        for j in range(n):
            img[b0 + j] = pos2
        pltpu.make_async_copy(
            img.at[pl.ds(b0, n)], o_hbm.at[pl.ds(b0, n)], sems.at[k % _NSEM]
        ).start()
    for k, (b0, n) in enumerate(groups):
        pltpu.make_async_copy(
            img.at[pl.ds(b0, n)], o_hbm.at[pl.ds(b0, n)], sems.at[k % _NSEM]
        ).wait()


def kernel(x, row_table, col_table):
    B, E, H, W = x.shape
    out3 = pl.pallas_call(
        _tc_body,
        grid=(1,),
        in_specs=[
            pl.BlockSpec((H, E), lambda i: (0, 0)),
            pl.BlockSpec((W, E), lambda i: (0, 0)),
        ],
        out_specs=pl.BlockSpec(memory_space=pltpu.MemorySpace.HBM),
        out_shape=jax.ShapeDtypeStruct((B, H * W, E), jnp.float32),
        scratch_shapes=[
            pltpu.VMEM((B, H * W, E), jnp.float32),
            pltpu.SemaphoreType.DMA((_NSEM,)),
        ],
    )(row_table, col_table)
    return jnp.transpose(out3.reshape(B, H, W, E), (0, 3, 1, 2))


# speedup vs baseline: 1.0077x; 1.0077x over previous
import jax
import jax.numpy as jnp
from jax.experimental import pallas as pl
from jax.experimental.pallas import tpu as pltpu

_B, _E, _H, _W = 32, 768, 16, 16
_HW = _H * _W
_GRP = 4
_NSEM = 8


def _tc_body(row_ref, col_ref, o_hbm, img, sems):
    row16 = row_ref[...]
    col16 = col_ref[...]
    pos2 = jnp.repeat(row16, _W, axis=0) + jnp.tile(col16, (_H, 1))
    groups = [(0, 1), (1, 3), (4, 4)] + [(b, 8) for b in range(8, _B, 8)]
    for k, (b0, n) in enumerate(groups):
        for j in range(n):
            img[b0 + j] = pos2
        pltpu.make_async_copy(
            img.at[pl.ds(b0, n)], o_hbm.at[pl.ds(b0, n)], sems.at[k % _NSEM]
        ).start()
    for k, (b0, n) in enumerate(groups):
        pltpu.make_async_copy(
            img.at[pl.ds(b0, n)], o_hbm.at[pl.ds(b0, n)], sems.at[k % _NSEM]
        ).wait()


def kernel(x, row_table, col_table):
    B, E, H, W = x.shape
    out3 = pl.pallas_call(
        _tc_body,
        grid=(1,),
        in_specs=[
            pl.BlockSpec((H, E), lambda i: (0, 0)),
            pl.BlockSpec((W, E), lambda i: (0, 0)),
        ],
        out_specs=pl.BlockSpec(memory_space=pltpu.MemorySpace.HBM),
        out_shape=jax.ShapeDtypeStruct((B, H * W, E), jnp.float32),
        scratch_shapes=[
            pltpu.VMEM((B, H * W, E), jnp.float32),
            pltpu.SemaphoreType.DMA((_NSEM,)),
        ],
    )(row_table, col_table)
    return jnp.transpose(out3.reshape(B, H, W, E), (0, 3, 1, 2))


# speedup vs baseline: 1.0152x; 1.0074x over previous
import jax
import jax.numpy as jnp
from jax.experimental import pallas as pl
from jax.experimental.pallas import tpu as pltpu

_B, _E, _H, _W = 32, 768, 16, 16
_HW = _H * _W
_GRP = 4
_NSEM = 8


def _tc_body(row_ref, col_ref, o_hbm, img, sems):
    row16 = row_ref[...]
    col16 = col_ref[...]
    pos2 = jnp.repeat(row16, _W, axis=0) + jnp.tile(col16, (_H, 1))
    groups = [(0, 1), (1, _GRP - 1)] + [(b, _GRP) for b in range(_GRP, _B, _GRP)]
    for k, (b0, n) in enumerate(groups):
        for j in range(n):
            img[b0 + j] = pos2
        pltpu.make_async_copy(
            img.at[pl.ds(b0, n)], o_hbm.at[pl.ds(b0, n)], sems.at[k % _NSEM]
        ).start()
    for k, (b0, n) in enumerate(groups):
        pltpu.make_async_copy(
            img.at[pl.ds(b0, n)], o_hbm.at[pl.ds(b0, n)], sems.at[k % _NSEM]
        ).wait()


def kernel(x, row_table, col_table):
    B, E, H, W = x.shape
    out3 = pl.pallas_call(
        _tc_body,
        grid=(1,),
        in_specs=[
            pl.BlockSpec((H, E), lambda i: (0, 0)),
            pl.BlockSpec((W, E), lambda i: (0, 0)),
        ],
        out_specs=pl.BlockSpec(memory_space=pltpu.MemorySpace.HBM),
        out_shape=jax.ShapeDtypeStruct((B, H * W, E), jnp.float32),
        scratch_shapes=[
            pltpu.VMEM((B, H * W, E), jnp.float32),
            pltpu.SemaphoreType.DMA((_NSEM,)),
        ],
    )(row_table, col_table)
    return jnp.transpose(out3.reshape(B, H, W, E), (0, 3, 1, 2))
